# baseline (speedup 1.0000x reference)
import functools

import jax
import jax.numpy as jnp
from jax import lax
from jax.experimental import pallas as pl
from jax.experimental.pallas import tpu as pltpu
from jax.experimental.pallas import tpu_sc as plsc


@functools.lru_cache(maxsize=None)
def _build_sc_kernel(q_len, k_len, in_dim, out_dim):
    info = plsc.get_sparse_core_info()
    num_cores, num_subcores, lanes = (
        info.num_cores, info.num_subcores, info.num_lanes)
    num_workers = num_cores * num_subcores
    rows_per_worker = q_len // num_workers
    window = k_len + rows_per_worker - 1
    window_pad = ((window + lanes - 1) // lanes) * lanes
    max_index = (in_dim - 1) // 2

    mesh = plsc.VectorSubcoreMesh(core_axis_name="c", subcore_axis_name="s")

    @functools.partial(
        pl.kernel,
        mesh=mesh,
        compiler_params=pltpu.CompilerParams(use_tc_tiling_on_sc=False),
        out_type=jax.ShapeDtypeStruct((q_len, k_len, out_dim), jnp.float32),
        scratch_types=[
            pltpu.VMEM((window_pad,), jnp.int32),
            pltpu.VMEM((window_pad, out_dim), jnp.float32),
            pltpu.SemaphoreType.DMA,
        ],
    )
    def rel_pos_kernel(emb_hbm, out_hbm, idx_v, yw_v, sem):
        wid = lax.axis_index("s") * num_cores + lax.axis_index("c")
        i0 = wid * rows_per_worker

        def build_idx(t, carry):
            m = t * lanes + lax.iota(jnp.int32, lanes)
            v = (rows_per_worker - 1) + i0 - m
            v = jnp.clip(v, -max_index, max_index) + max_index
            idx_v[pl.ds(t * lanes, lanes)] = v
            return carry

        lax.fori_loop(0, window_pad // lanes, build_idx, 0)

        pltpu.async_copy(emb_hbm.at[idx_v], yw_v, sem).wait()

        lag = 8
        handles = []
        for r in range(rows_per_worker):
            h = pltpu.async_copy(
                yw_v.at[pl.ds((rows_per_worker - 1) - r, k_len)],
                out_hbm.at[i0 + r],
                sem,
            )
            handles.append(h)
            if r >= lag:
                handles[r - lag].wait()
        for h in handles[rows_per_worker - lag:]:
            h.wait()

    return rel_pos_kernel


def kernel(q, k, embeddings):
    q_len = q.shape[1]
    k_len = k.shape[1]
    in_dim, out_dim = embeddings.shape
    return _build_sc_kernel(q_len, k_len, in_dim, out_dim)(embeddings)


# speedup vs baseline: 3.3734x; 3.3734x over previous
import functools

import jax
import jax.numpy as jnp
from jax import lax
from jax.experimental import pallas as pl
from jax.experimental.pallas import tpu as pltpu
from jax.experimental.pallas import tpu_sc as plsc


@functools.lru_cache(maxsize=None)
def _build_sc_kernel(q_len, k_len, in_dim, out_dim):
    info = plsc.get_sparse_core_info()
    num_cores, num_subcores, lanes = (
        info.num_cores, info.num_subcores, info.num_lanes)
    num_workers = num_cores * num_subcores
    rows_per_worker = q_len // num_workers
    phases = 8
    regions = num_workers // phases
    region_rows = q_len // regions
    stride_span = phases * (rows_per_worker - 1)
    window = k_len + stride_span
    window_pad = ((window + lanes - 1) // lanes) * lanes
    max_index = (in_dim - 1) // 2

    mesh = plsc.VectorSubcoreMesh(core_axis_name="c", subcore_axis_name="s")

    @functools.partial(
        pl.kernel,
        mesh=mesh,
        compiler_params=pltpu.CompilerParams(
            use_tc_tiling_on_sc=False, needs_layout_passes=False),
        out_type=jax.ShapeDtypeStruct((q_len, out_dim, k_len), jnp.float32),
        scratch_types=[
            pltpu.VMEM((out_dim * in_dim,), jnp.float32),
            pltpu.VMEM((out_dim, window_pad), jnp.float32),
            pltpu.SemaphoreType.DMA,
        ],
    )
    def rel_pos_kernel(emb_t_hbm, out_hbm, et_v, ywt_v, sem):
        wid = lax.axis_index("s") * num_cores + lax.axis_index("c")
        region = wid // phases
        phase = wid - region * phases
        ibase = region * region_rows + phase

        pltpu.sync_copy(emb_t_hbm, et_v)

        def build_chunk(t, carry):
            m = t * lanes + lax.iota(jnp.int32, lanes)
            v = stride_span + ibase - m
            col = jnp.clip(v, -max_index, max_index) + max_index
            for e in range(out_dim):
                vals = plsc.load_gather(et_v, [col + e * in_dim])
                ywt_v[e, pl.ds(t * lanes, lanes)] = vals
            return carry

        lax.fori_loop(0, window_pad // lanes, build_chunk, 0)

        lag = 8
        handles = []
        for t in range(rows_per_worker):
            h = pltpu.async_copy(
                ywt_v.at[:, pl.ds(stride_span - phases * t, k_len)],
                out_hbm.at[ibase + phases * t],
                sem,
            )
            handles.append(h)
            if t >= lag:
                handles[t - lag].wait()
        for h in handles[rows_per_worker - lag:]:
            h.wait()

    return rel_pos_kernel


def kernel(q, k, embeddings):
    q_len = q.shape[1]
    k_len = k.shape[1]
    in_dim, out_dim = embeddings.shape
    out_t = _build_sc_kernel(q_len, k_len, in_dim, out_dim)(
        embeddings.T.reshape(-1))
    return jnp.swapaxes(out_t, 1, 2)
